# baseline (speedup 1.0000x reference)
import functools

import jax
import jax.numpy as jnp
from jax import lax
from jax.experimental import pallas as pl
from jax.experimental.pallas import tpu as pltpu
from jax.experimental.pallas import tpu_sc as plsc

_V = 100000
_D = 64
_NI = 4096
_NJ = 200
_B = _NI * _NJ


def _idx_body(x_ref, idx_ref):
    v = (jnp.tanh(x_ref[...]) + 1.0) * (float(_V) / 2.0)
    idx_ref[...] = jnp.minimum(v.astype(jnp.int32), _V - 1)


def _compute_idx(x):
    return pl.pallas_call(
        _idx_body,
        out_shape=jax.ShapeDtypeStruct(x.shape, jnp.int32),
    )(x)


def _emb_body(e_ref, o_ref):
    a = e_ref[...].reshape(400, 2, _D)
    o_ref[...] = jnp.concatenate([a[:, 0, :], a[:, 1, :]], axis=-1)


def _compact_emb(emb):
    return pl.pallas_call(
        _emb_body,
        grid=(125,),
        in_specs=[pl.BlockSpec((800, _D), lambda g: (g, 0))],
        out_specs=pl.BlockSpec((400, 128), lambda g: (g, 0)),
        out_shape=jax.ShapeDtypeStruct((_V // 2, 128), jnp.float32),
    )(emb)


def _out_body(r_ref, o_ref):
    a = r_ref[...].reshape(8, 100, 128)
    lo = a[:, :, :_D]
    hi = a[:, :, _D:]
    o_ref[...] = jnp.stack([lo, hi], axis=2).reshape(8, 2 * 100, _D)


def _final(rows2):
    BI = 8
    return pl.pallas_call(
        _out_body,
        grid=(_NI // BI,),
        in_specs=[pl.BlockSpec((BI * _NJ // 2, 128), lambda g: (g, 0))],
        out_specs=pl.BlockSpec((BI, _NJ, _D), lambda g: (g, 0, 0)),
        out_shape=jax.ShapeDtypeStruct((_NI, _NJ, _D), jnp.float32),
    )(rows2)


@functools.cache
def _make_gather(B, D):
    info = plsc.get_sparse_core_info()
    nc, ns = info.num_cores, info.num_subcores
    nw = nc * ns
    b_per_w = B // nw
    CH = 128
    n_ch = b_per_w // CH
    NBUF = 4
    LOOK = 2
    assert nw * b_per_w == B and CH * n_ch == b_per_w
    assert n_ch % NBUF == 0 and n_ch >= NBUF + LOOK
    n_groups = n_ch // NBUF
    mesh = plsc.VectorSubcoreMesh(core_axis_name="c", subcore_axis_name="s")

    @functools.partial(
        pl.kernel,
        mesh=mesh,
        out_type=jax.ShapeDtypeStruct((B, D), jnp.float32),
        scratch_types=[
            pltpu.VMEM((n_ch, CH), jnp.int32),
            pltpu.VMEM((NBUF, CH, D), jnp.float32),
        ]
        + [pltpu.SemaphoreType.DMA] * (2 * NBUF),
        compiler_params=pltpu.CompilerParams(use_tc_tiling_on_sc=False),
    )
    def gather(emb_hbm, idx_hbm, out_hbm, idx_v, rows_v, *sems):
        G = sems[:NBUF]
        W = sems[NBUF:]
        wid = lax.axis_index("s") * nc + lax.axis_index("c")
        base = wid * b_per_w
        pltpu.sync_copy(idx_hbm.at[wid], idx_v)

        def chunk_step(c, j, fire_pre, wait_wb):
            if fire_pre:
                j2 = (j + LOOK) % NBUF
                if wait_wb:
                    pltpu.make_async_copy(
                        rows_v.at[j2], out_hbm.at[pl.ds(base, CH)], W[j2]
                    ).wait()
                pltpu.make_async_copy(
                    emb_hbm.at[idx_v.at[c + LOOK]], rows_v.at[j2], G[j2]
                ).start()
            pltpu.make_async_copy(
                emb_hbm.at[idx_v.at[c]], rows_v.at[j], G[j]
            ).wait()
            pltpu.make_async_copy(
                rows_v.at[j], out_hbm.at[pl.ds(base + c * CH, CH)], W[j]
            ).start()

        for c in range(LOOK):
            pltpu.make_async_copy(
                emb_hbm.at[idx_v.at[c]], rows_v.at[c], G[c]
            ).start()
        for j in range(NBUF):
            chunk_step(j, j, fire_pre=j + LOOK < n_ch, wait_wb=j + LOOK >= NBUF)

        def group(g, carry):
            c0 = g * NBUF
            for j in range(NBUF):
                chunk_step(c0 + j, j, fire_pre=True, wait_wb=True)
            return carry

        lax.fori_loop(1, n_groups - 1, group, 0)

        for j in range(NBUF):
            c = n_ch - NBUF + j
            chunk_step(c, j, fire_pre=c + LOOK < n_ch, wait_wb=c + LOOK < n_ch)
        for c in range(n_ch - NBUF, n_ch):
            pltpu.make_async_copy(
                rows_v.at[c % NBUF], out_hbm.at[pl.ds(base, CH)], W[c % NBUF]
            ).wait()

    return gather


def kernel(x, emb):
    idx = _compute_idx(x).reshape(32, _B // 32 // 128, 128)
    emb_lin = _compact_emb(emb).reshape(_V, _D)
    out_lin = _make_gather(_B, _D)(emb_lin, idx)
    return _final(out_lin.reshape(_B // 2, 128))


# speedup vs baseline: 1.6130x; 1.6130x over previous
import functools

import jax
import jax.numpy as jnp
from jax import lax
from jax.experimental import pallas as pl
from jax.experimental.pallas import tpu as pltpu
from jax.experimental.pallas import tpu_sc as plsc

_V = 100000
_D = 64
_NI = 4096
_NJ = 200
_B = _NI * _NJ


def _idx_body(x_ref, idx_ref):
    v = (jnp.tanh(x_ref[...]) + 1.0) * (float(_V) / 2.0)
    idx_ref[...] = jnp.minimum(v.astype(jnp.int32), _V - 1)


def _compute_idx(x):
    return pl.pallas_call(
        _idx_body,
        out_shape=jax.ShapeDtypeStruct(x.shape, jnp.int32),
    )(x)


@functools.cache
def _make_gather(B, D):
    info = plsc.get_sparse_core_info()
    nc, ns = info.num_cores, info.num_subcores
    nw = nc * ns
    b_per_w = B // nw
    CH = 128
    n_ch = b_per_w // CH
    NBUF = 4
    LOOK = 2
    assert nw * b_per_w == B and CH * n_ch == b_per_w
    assert n_ch % NBUF == 0 and n_ch >= NBUF + LOOK
    n_groups = n_ch // NBUF
    mesh = plsc.VectorSubcoreMesh(core_axis_name="c", subcore_axis_name="s")

    @functools.partial(
        pl.kernel,
        mesh=mesh,
        out_type=jax.ShapeDtypeStruct((B, D), jnp.float32),
        scratch_types=[
            pltpu.VMEM((n_ch, CH), jnp.int32),
            pltpu.VMEM((NBUF, CH, D), jnp.float32),
        ]
        + [pltpu.SemaphoreType.DMA] * (2 * NBUF),
        compiler_params=pltpu.CompilerParams(use_tc_tiling_on_sc=False),
    )
    def gather(emb_hbm, idx_hbm, out_hbm, idx_v, rows_v, *sems):
        G = sems[:NBUF]
        W = sems[NBUF:]
        wid = lax.axis_index("s") * nc + lax.axis_index("c")
        base = wid * b_per_w
        pltpu.sync_copy(idx_hbm.at[wid], idx_v)

        def chunk_step(c, j, fire_pre, wait_wb):
            if fire_pre:
                j2 = (j + LOOK) % NBUF
                if wait_wb:
                    pltpu.make_async_copy(
                        rows_v.at[j2], out_hbm.at[pl.ds(base, CH)], W[j2]
                    ).wait()
                pltpu.make_async_copy(
                    emb_hbm.at[idx_v.at[c + LOOK]], rows_v.at[j2], G[j2]
                ).start()
            pltpu.make_async_copy(
                emb_hbm.at[idx_v.at[c]], rows_v.at[j], G[j]
            ).wait()
            pltpu.make_async_copy(
                rows_v.at[j], out_hbm.at[pl.ds(base + c * CH, CH)], W[j]
            ).start()

        for c in range(LOOK):
            pltpu.make_async_copy(
                emb_hbm.at[idx_v.at[c]], rows_v.at[c], G[c]
            ).start()
        for j in range(NBUF):
            chunk_step(j, j, fire_pre=j + LOOK < n_ch, wait_wb=j + LOOK >= NBUF)

        def group(g, carry):
            c0 = g * NBUF
            for j in range(NBUF):
                chunk_step(c0 + j, j, fire_pre=True, wait_wb=True)
            return carry

        lax.fori_loop(1, n_groups - 1, group, 0)

        for j in range(NBUF):
            c = n_ch - NBUF + j
            chunk_step(c, j, fire_pre=c + LOOK < n_ch, wait_wb=c + LOOK < n_ch)
        for c in range(n_ch - NBUF, n_ch):
            pltpu.make_async_copy(
                rows_v.at[c % NBUF], out_hbm.at[pl.ds(base, CH)], W[c % NBUF]
            ).wait()

    return gather


def kernel(x, emb):
    idx = _compute_idx(x).reshape(32, _B // 32 // 128, 128)
    out_lin = _make_gather(_B, _D)(emb, idx)
    return out_lin.reshape(_NI, _NJ, _D)


# speedup vs baseline: 1.6180x; 1.0031x over previous
import functools

import jax
import jax.numpy as jnp
from jax import lax
from jax.experimental import pallas as pl
from jax.experimental.pallas import tpu as pltpu
from jax.experimental.pallas import tpu_sc as plsc

_V = 100000
_D = 64
_NI = 4096
_NJ = 200
_B = _NI * _NJ


def _idx_body(x_ref, idx_ref):
    v = (jnp.tanh(x_ref[...]) + 1.0) * (float(_V) / 2.0)
    idx_ref[...] = jnp.minimum(v.astype(jnp.int32), _V - 1)


def _compute_idx(x):
    return pl.pallas_call(
        _idx_body,
        out_shape=jax.ShapeDtypeStruct(x.shape, jnp.int32),
    )(x)


@functools.cache
def _make_gather(B, D):
    info = plsc.get_sparse_core_info()
    nc, ns = info.num_cores, info.num_subcores
    nw = nc * ns
    b_per_w = B // nw
    CH = 128
    n_ch = b_per_w // CH
    NBUF = 4
    LOOK = 3
    assert nw * b_per_w == B and CH * n_ch == b_per_w
    assert n_ch % NBUF == 0 and n_ch >= NBUF + LOOK
    n_groups = n_ch // NBUF
    mesh = plsc.VectorSubcoreMesh(core_axis_name="c", subcore_axis_name="s")

    @functools.partial(
        pl.kernel,
        mesh=mesh,
        out_type=jax.ShapeDtypeStruct((B, D), jnp.float32),
        scratch_types=[
            pltpu.VMEM((n_ch, CH), jnp.int32),
            pltpu.VMEM((NBUF, CH, D), jnp.float32),
        ]
        + [pltpu.SemaphoreType.DMA] * (2 * NBUF),
        compiler_params=pltpu.CompilerParams(use_tc_tiling_on_sc=False),
    )
    def gather(emb_hbm, idx_hbm, out_hbm, idx_v, rows_v, *sems):
        G = sems[:NBUF]
        W = sems[NBUF:]
        wid = lax.axis_index("s") * nc + lax.axis_index("c")
        base = wid * b_per_w
        pltpu.sync_copy(idx_hbm.at[wid], idx_v)

        def chunk_step(c, j, fire_pre, wait_wb):
            if fire_pre:
                j2 = (j + LOOK) % NBUF
                if wait_wb:
                    pltpu.make_async_copy(
                        rows_v.at[j2], out_hbm.at[pl.ds(base, CH)], W[j2]
                    ).wait()
                pltpu.make_async_copy(
                    emb_hbm.at[idx_v.at[c + LOOK]], rows_v.at[j2], G[j2]
                ).start()
            pltpu.make_async_copy(
                emb_hbm.at[idx_v.at[c]], rows_v.at[j], G[j]
            ).wait()
            pltpu.make_async_copy(
                rows_v.at[j], out_hbm.at[pl.ds(base + c * CH, CH)], W[j]
            ).start()

        for c in range(LOOK):
            pltpu.make_async_copy(
                emb_hbm.at[idx_v.at[c]], rows_v.at[c], G[c]
            ).start()
        for j in range(NBUF):
            chunk_step(j, j, fire_pre=j + LOOK < n_ch, wait_wb=j + LOOK >= NBUF)

        def group(g, carry):
            c0 = g * NBUF
            for j in range(NBUF):
                chunk_step(c0 + j, j, fire_pre=True, wait_wb=True)
            return carry

        lax.fori_loop(1, n_groups - 1, group, 0)

        for j in range(NBUF):
            c = n_ch - NBUF + j
            chunk_step(c, j, fire_pre=c + LOOK < n_ch, wait_wb=c + LOOK < n_ch)
        for c in range(n_ch - NBUF, n_ch):
            pltpu.make_async_copy(
                rows_v.at[c % NBUF], out_hbm.at[pl.ds(base, CH)], W[c % NBUF]
            ).wait()

    return gather


def kernel(x, emb):
    idx = _compute_idx(x).reshape(32, _B // 32 // 128, 128)
    out_lin = _make_gather(_B, _D)(emb, idx)
    return out_lin.reshape(_NI, _NJ, _D)
